# baseline (speedup 1.0000x reference)
import functools

import jax
import jax.numpy as jnp
from jax import lax
from jax.experimental import pallas as pl
from jax.experimental.pallas import tpu as pltpu
from jax.experimental.pallas import tpu_sc as plsc

N_PROPS = 528
D_HALF = 256
D_FULL = 512
PE_ROWS = 64
ROWS_PER_WORKER = 24
N_ACTIVE_WORKERS = N_PROPS // ROWS_PER_WORKER

NP = 16
NB = 8
N_CHUNKS = 8

_SC_MESH = plsc.VectorSubcoreMesh(core_axis_name="c", subcore_axis_name="s")


@functools.partial(
    pl.kernel,
    mesh=_SC_MESH,
    out_type=(
        jax.ShapeDtypeStruct((N_PROPS, D_HALF), jnp.float32),
        jax.ShapeDtypeStruct((32, NB, NP, D_FULL), jnp.float32),
    ),
    scratch_types=[
        pltpu.VMEM((ROWS_PER_WORKER,), jnp.int32),
        pltpu.VMEM((ROWS_PER_WORKER, D_HALF), jnp.float32),
        pltpu.VMEM((NB, NP, D_FULL), jnp.float32),
        pltpu.SemaphoreType.DMA,
    ],
)
def _sc_gather_bias(table_hbm, idx_hbm, x_hbm, out_hbm, probe_hbm,
                    idx_v, rows_v, xbuf, sem):
    wid = lax.axis_index("s") * 2 + lax.axis_index("c")

    @pl.when(wid < N_ACTIVE_WORKERS)
    def _():
        base = wid * ROWS_PER_WORKER
        pltpu.sync_copy(idx_hbm.at[pl.ds(base, ROWS_PER_WORKER)], idx_v)
        pltpu.async_copy(table_hbm.at[idx_v], rows_v, sem).wait()
        pltpu.sync_copy(rows_v, out_hbm.at[pl.ds(base, ROWS_PER_WORKER)])

    p0 = wid * NP

    def body(g, carry):
        pltpu.sync_copy(
            x_hbm.at[pl.ds(g * NB, NB), pl.ds(p0, NP), :], xbuf)
        return carry

    lax.fori_loop(0, N_CHUNKS, body, 0)
    pltpu.sync_copy(xbuf, probe_hbm.at[wid])


def _add_body(x_ref, b_ref, o_ref):
    b = b_ref[...]
    o_ref[:, :, :D_HALF] = x_ref[:, :, :D_HALF] + b[None]
    o_ref[:, :, D_HALF:] = x_ref[:, :, D_HALF:] + b[None]


def kernel(x, pe, props):
    bsz = x.shape[0]
    table = pe.reshape(PE_ROWS, D_HALF)
    idx = props[:, 0]

    bias, _probe = _sc_gather_bias(table, idx, x)

    b_blk = 8
    out = pl.pallas_call(
        _add_body,
        grid=(bsz // b_blk,),
        in_specs=[
            pl.BlockSpec((b_blk, N_PROPS, D_FULL), lambda i: (i, 0, 0)),
            pl.BlockSpec((N_PROPS, D_HALF), lambda i: (0, 0)),
        ],
        out_specs=pl.BlockSpec((b_blk, N_PROPS, D_FULL), lambda i: (i, 0, 0)),
        out_shape=jax.ShapeDtypeStruct(x.shape, x.dtype),
        compiler_params=pltpu.CompilerParams(
            dimension_semantics=("parallel",),
        ),
    )(x, bias)
    return out


# speedup vs baseline: 1.1388x; 1.1388x over previous
import functools

import jax
import jax.numpy as jnp
from jax import lax
from jax.experimental import pallas as pl
from jax.experimental.pallas import tpu as pltpu
from jax.experimental.pallas import tpu_sc as plsc

N_PROPS = 528
D_HALF = 256
D_FULL = 512
PE_ROWS = 64
ROWS_PER_WORKER = 24
N_CHUNKS = N_PROPS // ROWS_PER_WORKER

_SC_MESH = plsc.VectorSubcoreMesh(
    core_axis_name="c", subcore_axis_name="s", num_cores=1)


@functools.partial(
    pl.kernel,
    mesh=_SC_MESH,
    out_type=jax.ShapeDtypeStruct((N_PROPS, D_HALF), jnp.float32),
    scratch_types=[
        pltpu.VMEM((ROWS_PER_WORKER,), jnp.int32),
        pltpu.VMEM((ROWS_PER_WORKER, D_HALF), jnp.float32),
        pltpu.SemaphoreType.DMA,
    ],
)
def _sc_gather_bias(table_hbm, idx_hbm, out_hbm, idx_v, rows_v, sem):
    wid = lax.axis_index("s")

    def do_chunk(base):
        pltpu.sync_copy(idx_hbm.at[pl.ds(base, ROWS_PER_WORKER)], idx_v)
        pltpu.async_copy(table_hbm.at[idx_v], rows_v, sem).wait()
        pltpu.sync_copy(rows_v, out_hbm.at[pl.ds(base, ROWS_PER_WORKER)])

    do_chunk(wid * ROWS_PER_WORKER)

    @pl.when(wid < N_CHUNKS - 16)
    def _():
        do_chunk((16 + wid) * ROWS_PER_WORKER)


def _add_body(x_ref, b_ref, o_ref):
    b = b_ref[...]
    o_ref[:, :, :D_HALF] = x_ref[:, :, :D_HALF] + b[None]
    o_ref[:, :, D_HALF:] = x_ref[:, :, D_HALF:] + b[None]


def kernel(x, pe, props):
    bsz = x.shape[0]
    table = pe.reshape(PE_ROWS, D_HALF)
    idx = props[:, 0]

    bias = _sc_gather_bias(table, idx)

    b_blk = 8
    out = pl.pallas_call(
        _add_body,
        grid=(bsz // b_blk,),
        in_specs=[
            pl.BlockSpec((b_blk, N_PROPS, D_FULL), lambda i: (i, 0, 0)),
            pl.BlockSpec((N_PROPS, D_HALF), lambda i: (0, 0)),
        ],
        out_specs=pl.BlockSpec((b_blk, N_PROPS, D_FULL), lambda i: (i, 0, 0)),
        out_shape=jax.ShapeDtypeStruct(x.shape, x.dtype),
        compiler_params=pltpu.CompilerParams(
            dimension_semantics=("parallel",),
        ),
    )(x, bias)
    return out


# speedup vs baseline: 1.1645x; 1.0226x over previous
import functools

import jax
import jax.numpy as jnp
from jax import lax
from jax.experimental import pallas as pl
from jax.experimental.pallas import tpu as pltpu
from jax.experimental.pallas import tpu_sc as plsc

N_PROPS = 528
D_HALF = 256
D_FULL = 512
PE_ROWS = 64
ROWS_PER_WORKER = 24
N_ACTIVE_WORKERS = N_PROPS // ROWS_PER_WORKER

_SC_MESH = plsc.VectorSubcoreMesh(core_axis_name="c", subcore_axis_name="s")


@functools.partial(
    pl.kernel,
    mesh=_SC_MESH,
    out_type=jax.ShapeDtypeStruct((N_PROPS, D_HALF), jnp.float32),
    scratch_types=[
        pltpu.VMEM((ROWS_PER_WORKER,), jnp.int32),
        pltpu.VMEM((ROWS_PER_WORKER, D_HALF), jnp.float32),
        pltpu.SemaphoreType.DMA,
    ],
)
def _sc_gather_bias(table_hbm, idx_hbm, out_hbm, idx_v, rows_v, sem):
    wid = lax.axis_index("s") * 2 + lax.axis_index("c")

    @pl.when(wid < N_ACTIVE_WORKERS)
    def _():
        base = wid * ROWS_PER_WORKER
        pltpu.sync_copy(idx_hbm.at[pl.ds(base, ROWS_PER_WORKER)], idx_v)
        pltpu.async_copy(table_hbm.at[idx_v], rows_v, sem).wait()
        pltpu.sync_copy(rows_v, out_hbm.at[pl.ds(base, ROWS_PER_WORKER)])


def _add_body(x_ref, pe_ref, left_ref, o_ref, bias_ref):
    @pl.when(pl.program_id(0) == 0)
    def _():
        left = left_ref[...]
        iota = lax.broadcasted_iota(jnp.int32, (N_PROPS, PE_ROWS), 1)
        onehot = jnp.where(left == iota, 1.0, 0.0).astype(jnp.float32)
        bias_ref[...] = jnp.dot(onehot, pe_ref[...],
                                preferred_element_type=jnp.float32)

    b = bias_ref[...]
    o_ref[:, :, :D_HALF] = x_ref[:, :, :D_HALF] + b[None]
    o_ref[:, :, D_HALF:] = x_ref[:, :, D_HALF:] + b[None]


def kernel(x, pe, props):
    bsz = x.shape[0]
    table = pe.reshape(PE_ROWS, D_HALF)
    idx = props[:, 0]
    left = props[:, :1]

    bias_sc = _sc_gather_bias(table, idx)

    b_blk = 8
    out = pl.pallas_call(
        _add_body,
        grid=(bsz // b_blk,),
        in_specs=[
            pl.BlockSpec((b_blk, N_PROPS, D_FULL), lambda i: (i, 0, 0)),
            pl.BlockSpec((PE_ROWS, D_HALF), lambda i: (0, 0)),
            pl.BlockSpec((N_PROPS, 1), lambda i: (0, 0)),
        ],
        out_specs=pl.BlockSpec((b_blk, N_PROPS, D_FULL), lambda i: (i, 0, 0)),
        out_shape=jax.ShapeDtypeStruct(x.shape, x.dtype),
        scratch_shapes=[pltpu.VMEM((N_PROPS, D_HALF), jnp.float32)],
        compiler_params=pltpu.CompilerParams(
            dimension_semantics=("arbitrary",),
        ),
    )(x, table, left)

    out = out.at[0, :, :D_HALF].set(x[0, :, :D_HALF] + bias_sc)
    return out
